# baseline (speedup 1.0000x reference)
import functools

import jax
import jax.numpy as jnp
from jax import lax
from jax.experimental import pallas as pl
from jax.experimental.pallas import tpu as pltpu
from jax.experimental.pallas import tpu_sc as plsc

_EPS = 0.5
_N = 4096
_NC = 2
_NS = 16
_NW = _NC * _NS
_ROWS_PER_W = _N // _NW
_CHUNK = 4
_DEPTH = 3
_NCHUNK = _ROWS_PER_W // _CHUNK
_LANES = 16
_UNROLL = 1

_mesh = plsc.VectorSubcoreMesh(core_axis_name="c", subcore_axis_name="s")

_scratch = (
    [pltpu.VMEM((_CHUNK, _N), jnp.float32) for _ in range(2 * _DEPTH)]
    + [pltpu.SemaphoreType.DMA for _ in range(2 * _DEPTH)]
)


@functools.partial(
    pl.kernel,
    out_type=jax.ShapeDtypeStruct((_N, _N), jnp.float32),
    mesh=_mesh,
    scratch_types=_scratch,
)
def _sc_mask(adj_hbm, out_hbm, *bufs_and_sems):
    ibufs = bufs_and_sems[:_DEPTH]
    obufs = bufs_and_sems[_DEPTH : 2 * _DEPTH]
    isems = bufs_and_sems[2 * _DEPTH : 3 * _DEPTH]
    osems = bufs_and_sems[3 * _DEPTH :]

    wid = lax.axis_index("s") * _NC + lax.axis_index("c")
    base = wid * _ROWS_PER_W

    def start_in(k):
        b = k % _DEPTH
        return pltpu.async_copy(
            adj_hbm.at[pl.ds(base + k * _CHUNK, _CHUNK)], ibufs[b], isems[b]
        )

    _HALF = _CHUNK // 2

    def compute_half(b, h):
        def body(j, carry):
            c = j * _LANES
            for r in range(h * _HALF, (h + 1) * _HALF):
                v = ibufs[b][r, pl.ds(c, _LANES)]
                obufs[b][r, pl.ds(c, _LANES)] = jnp.where(v > _EPS, v, 0.0)
            return carry

        lax.fori_loop(0, _N // _LANES, body, 0)

    cp_in = [start_in(k) for k in range(_DEPTH)]
    pending_out = [None] * _DEPTH
    for k in range(_NCHUNK):
        b = k % _DEPTH
        cp_in[b].wait()
        if pending_out[b] is not None:
            for cp in pending_out[b]:
                cp.wait()
        outs = []
        for h in range(2):
            compute_half(b, h)
            outs.append(
                pltpu.async_copy(
                    obufs[b].at[pl.ds(h * _HALF, _HALF)],
                    out_hbm.at[pl.ds(base + k * _CHUNK + h * _HALF, _HALF)],
                    osems[b],
                )
            )
        pending_out[b] = outs
        if k + _DEPTH < _NCHUNK:
            cp_in[b] = start_in(k + _DEPTH)
    for b in range(_DEPTH):
        if pending_out[b] is not None:
            for cp in pending_out[b]:
                cp.wait()


def kernel(adj):
    return _sc_mask(adj)


# speedup vs baseline: 1.3654x; 1.3654x over previous
import functools

import jax
import jax.numpy as jnp
from jax import lax
from jax.experimental import pallas as pl
from jax.experimental.pallas import tpu as pltpu
from jax.experimental.pallas import tpu_sc as plsc

_EPS = 0.5
_N = 4096
_NC = 2
_NS = 16
_NW = _NC * _NS
_ROWS_PER_W = _N // _NW
_CHUNK = 4
_DEPTH = 3
_NCHUNK = _ROWS_PER_W // _CHUNK
_LANES = 16

_mesh = plsc.VectorSubcoreMesh(core_axis_name="c", subcore_axis_name="s")

_scratch = (
    [pltpu.VMEM((_CHUNK, _N), jnp.float32) for _ in range(2 * _DEPTH)]
    + [pltpu.SemaphoreType.DMA for _ in range(2 * _DEPTH)]
)


@functools.partial(
    pl.kernel,
    out_type=jax.ShapeDtypeStruct((_N, _N), jnp.float32),
    mesh=_mesh,
    scratch_types=_scratch,
)
def _sc_mask(adj_hbm, out_hbm, *bufs_and_sems):
    ibufs = bufs_and_sems[:_DEPTH]
    obufs = bufs_and_sems[_DEPTH : 2 * _DEPTH]
    isems = bufs_and_sems[2 * _DEPTH : 3 * _DEPTH]
    osems = bufs_and_sems[3 * _DEPTH :]

    wid = lax.axis_index("s") * _NC + lax.axis_index("c")
    base = wid * _ROWS_PER_W

    def start_in(k):
        b = k % _DEPTH
        return pltpu.async_copy(
            adj_hbm.at[pl.ds(base + k * _CHUNK, _CHUNK)], ibufs[b], isems[b]
        )

    def compute(b):
        def body(j, carry):
            c = j * _LANES
            for r in range(_CHUNK):
                v = ibufs[b][r, pl.ds(c, _LANES)]
                obufs[b][r, pl.ds(c, _LANES)] = jnp.where(v > _EPS, v, 0.0)
            return carry

        lax.fori_loop(0, _N // _LANES, body, 0)

    cp_in = [start_in(k) for k in range(_DEPTH)]
    pending_out = [None] * _DEPTH
    for k in range(_NCHUNK):
        b = k % _DEPTH
        cp_in[b].wait()
        if pending_out[b] is not None:
            pending_out[b].wait()
        compute(b)
        pending_out[b] = pltpu.async_copy(
            obufs[b], out_hbm.at[pl.ds(base + k * _CHUNK, _CHUNK)], osems[b]
        )
        if k + _DEPTH < _NCHUNK:
            cp_in[b] = start_in(k + _DEPTH)
    for b in range(_DEPTH):
        if pending_out[b] is not None:
            pending_out[b].wait()


def kernel(adj):
    return _sc_mask(adj)


# speedup vs baseline: 1.3738x; 1.0062x over previous
import functools

import jax
import jax.numpy as jnp
from jax import lax
from jax.experimental import pallas as pl
from jax.experimental.pallas import tpu as pltpu
from jax.experimental.pallas import tpu_sc as plsc

_EPS = 0.5
_N = 4096
_NC = 2
_NS = 16
_NW = _NC * _NS
_ROWS_PER_W = _N // _NW
_CHUNK = 4
_IDEPTH = 4
_ODEPTH = 3
_NCHUNK = _ROWS_PER_W // _CHUNK
_LANES = 16

_mesh = plsc.VectorSubcoreMesh(core_axis_name="c", subcore_axis_name="s")

_scratch = (
    [pltpu.VMEM((_CHUNK, _N), jnp.float32) for _ in range(_IDEPTH + _ODEPTH)]
    + [pltpu.SemaphoreType.DMA for _ in range(_IDEPTH + _ODEPTH)]
)


@functools.partial(
    pl.kernel,
    out_type=jax.ShapeDtypeStruct((_N, _N), jnp.float32),
    mesh=_mesh,
    scratch_types=_scratch,
)
def _sc_mask(adj_hbm, out_hbm, *bufs_and_sems):
    nb = _IDEPTH + _ODEPTH
    ibufs = bufs_and_sems[:_IDEPTH]
    obufs = bufs_and_sems[_IDEPTH:nb]
    isems = bufs_and_sems[nb : nb + _IDEPTH]
    osems = bufs_and_sems[nb + _IDEPTH :]

    wid = lax.axis_index("s") * _NC + lax.axis_index("c")
    base = wid * _ROWS_PER_W

    def start_in(k):
        b = k % _IDEPTH
        return pltpu.async_copy(
            adj_hbm.at[pl.ds(base + k * _CHUNK, _CHUNK)], ibufs[b], isems[b]
        )

    def compute(bi, bo):
        def body(j, carry):
            c = j * _LANES
            for r in range(_CHUNK):
                v = ibufs[bi][r, pl.ds(c, _LANES)]
                obufs[bo][r, pl.ds(c, _LANES)] = jnp.where(v > _EPS, v, 0.0)
            return carry

        lax.fori_loop(0, _N // _LANES, body, 0)

    cp_in = [start_in(k) for k in range(_IDEPTH)]
    pending_out = [None] * _ODEPTH
    for k in range(_NCHUNK):
        bi = k % _IDEPTH
        bo = k % _ODEPTH
        cp_in[bi].wait()
        if pending_out[bo] is not None:
            pending_out[bo].wait()
        compute(bi, bo)
        pending_out[bo] = pltpu.async_copy(
            obufs[bo], out_hbm.at[pl.ds(base + k * _CHUNK, _CHUNK)], osems[bo]
        )
        if k + _IDEPTH < _NCHUNK:
            cp_in[bi] = start_in(k + _IDEPTH)
    for bo in range(_ODEPTH):
        if pending_out[bo] is not None:
            pending_out[bo].wait()


def kernel(adj):
    return _sc_mask(adj)
